# Initial kernel scaffold; baseline (speedup 1.0000x reference)
import jax
import jax.numpy as jnp
from jax.experimental import pallas as pl


def kernel(x_l0, x_l1, x_l2, x_l3, params, knn_idx):
    raise NotImplementedError("write your pallas kernel here")


# speedup vs baseline: 3.5267x; 3.5267x over previous
import functools

import jax
import jax.numpy as jnp
from jax import lax
from jax.experimental import pallas as pl
from jax.experimental.pallas import tpu as pltpu
from jax.experimental.pallas import tpu_sc as plsc

_N = 10000
_C = 256
_K = 16
_T = 400
_NT = _N // _T
_CH = 128
_NCHUNK = (_N * _K) // _CH
_NW = 32
_ITERS = (_NCHUNK + _NW - 1) // _NW
_RSQRT_C = 1.0 / (_C ** 0.5)
_GATE_PAD = -1e30


def _ln(x, g, b, eps=1e-5):
    mu = jnp.mean(x, axis=-1, keepdims=True)
    var = jnp.mean((x - mu) ** 2, axis=-1, keepdims=True)
    return (x - mu) * lax.rsqrt(var + eps) * g + b


def _proj_body(x0, x1, x2,
               wq0, bq0, wk0, wq1, bq1, wk1, wq2, bq2, wk2,
               qp0, qp1, qp2, qt0, qt1, qt2):
    for x, wq, bq, wk, qp, qt in (
        (x0, wq0, bq0, wk0, qp0, qt0),
        (x1, wq1, bq1, wk1, qp1, qt1),
        (x2, wq2, bq2, wk2, qp2, qt2),
    ):
        q = jnp.dot(x[...], wq[...], preferred_element_type=jnp.float32) + bq[...]
        qp[...] = q
        qt[...] = lax.dot_general(
            q, wk[...], (((1,), (1,)), ((), ())),
            preferred_element_type=jnp.float32) * _RSQRT_C


def _tc_proj(x0, x1, x2, blocks):
    row = pl.BlockSpec((_T, _C), lambda i: (i, 0))
    wspec = pl.BlockSpec((_C, _C), lambda i: (0, 0))
    bspec = pl.BlockSpec((1, _C), lambda i: (0, 0))
    out = jax.ShapeDtypeStruct((_N, _C), jnp.float32)
    args = [x0, x1, x2]
    in_specs = [row, row, row]
    for blk in blocks:
        args += [blk['Wq'], blk['bq'].reshape(1, _C), blk['Wk']]
        in_specs += [wspec, bspec, wspec]
    return pl.pallas_call(
        _proj_body,
        grid=(_NT,),
        in_specs=in_specs,
        out_specs=[row] * 6,
        out_shape=[out] * 6,
    )(*args)


def _sc_gather_body(knn_hbm, x_hbm, g_hbm, idx_v, rows_v, sem):
    wid = lax.axis_index("s") * 2 + lax.axis_index("c")

    def step(t, carry):
        ch = wid + _NW * t

        @pl.when(ch < _NCHUNK)
        def _():
            pltpu.sync_copy(knn_hbm.at[pl.ds(ch * _CH, _CH)], idx_v)
            pltpu.async_copy(x_hbm.at[idx_v], rows_v, sem).wait()
            pltpu.sync_copy(rows_v, g_hbm.at[pl.ds(ch * _CH, _CH)])

        return carry

    lax.fori_loop(0, _ITERS, step, 0)


def _sc_gather(knn_flat, x):
    mesh = plsc.VectorSubcoreMesh(core_axis_name="c", subcore_axis_name="s")
    f = functools.partial(
        pl.kernel,
        out_type=jax.ShapeDtypeStruct((_N * _K, _C), jnp.float32),
        mesh=mesh,
        scratch_types=[
            pltpu.VMEM((_CH,), jnp.int32),
            pltpu.VMEM((_CH, _C), jnp.float32),
            pltpu.SemaphoreType.DMA,
        ],
    )(_sc_gather_body)
    return f(knn_flat, x)


def _attn_body(g, qt0, qt1, qt2, qp0, qp1, qp2, x3,
               wv0, bv0, lg0, lb0, wv1, bv1, lg1, lb1, wv2, bv2, lg2, lb2,
               w1, b1, w2p, b2p, lgf, lbf, out):
    gv = g[...]
    x = x3[...]
    outs = []
    for qt, qp, wv, bv, lg, lb in (
        (qt0, qp0, wv0, bv0, lg0, lb0),
        (qt1, qp1, wv1, bv1, lg1, lb1),
        (qt2, qp2, wv2, bv2, lg2, lb2),
    ):
        q = qt[...]
        s = [jnp.sum(gv[:, k, :] * q, axis=-1, keepdims=True)
             for k in range(_K)]
        m = s[0]
        for k in range(1, _K):
            m = jnp.maximum(m, s[k])
        e = [jnp.exp(sk - m) for sk in s]
        denom = e[0]
        for k in range(1, _K):
            denom = denom + e[k]
        inv = 1.0 / denom
        sv = e[0] * gv[:, 0, :]
        for k in range(1, _K):
            sv = sv + e[k] * gv[:, k, :]
        sv = sv * inv
        o = jnp.dot(sv, wv[...], preferred_element_type=jnp.float32) + bv[...]
        outs.append(_ln(o + qp[...], lg[...], lb[...]))

    h = jnp.maximum(jnp.dot(x, w1[...], preferred_element_type=jnp.float32)
                    + b1[...], 0.0)
    logits = jnp.dot(h, w2p[...], preferred_element_type=jnp.float32) + b2p[...]
    lm = jnp.max(logits, axis=-1, keepdims=True)
    le = jnp.exp(logits - lm)
    linv = 1.0 / jnp.sum(le, axis=-1, keepdims=True)
    mix = (le[:, 0:1] * outs[0] + le[:, 1:2] * outs[1]
           + le[:, 2:3] * outs[2]) * linv
    out[...] = _ln(mix + x, lgf[...], lbf[...])


def _tc_attn(g, qts, qps, x3, params):
    row = pl.BlockSpec((_T, _C), lambda i: (i, 0))
    g3 = pl.BlockSpec((_T, _K, _C), lambda i: (i, 0, 0))
    wspec = pl.BlockSpec((_C, _C), lambda i: (0, 0))
    bspec = pl.BlockSpec((1, _C), lambda i: (0, 0))
    hspec = pl.BlockSpec((_C, _C // 2), lambda i: (0, 0))
    h1spec = pl.BlockSpec((1, _C // 2), lambda i: (0, 0))
    gspec = pl.BlockSpec((_C // 2, _C // 2), lambda i: (0, 0))
    g1spec = pl.BlockSpec((1, _C // 2), lambda i: (0, 0))

    gate = params['gate']
    w2p = jnp.zeros((_C // 2, _C // 2), jnp.float32).at[:, :3].set(gate['W2'])
    b2p = jnp.full((_C // 2,), _GATE_PAD, jnp.float32).at[:3].set(gate['b2'])

    args = [g, qts[0], qts[1], qts[2], qps[0], qps[1], qps[2], x3]
    in_specs = [g3, row, row, row, row, row, row, row]
    for blk in params['blocks']:
        args += [blk['Wv'], blk['bv'].reshape(1, _C),
                 blk['ln_g'].reshape(1, _C), blk['ln_b'].reshape(1, _C)]
        in_specs += [wspec, bspec, bspec, bspec]
    args += [gate['W1'], gate['b1'].reshape(1, _C // 2), w2p,
             b2p.reshape(1, _C // 2),
             params['ln_g'].reshape(1, _C), params['ln_b'].reshape(1, _C)]
    in_specs += [hspec, h1spec, gspec, g1spec, bspec, bspec]

    return pl.pallas_call(
        _attn_body,
        grid=(_NT,),
        in_specs=in_specs,
        out_specs=row,
        out_shape=jax.ShapeDtypeStruct((_N, _C), jnp.float32),
    )(*args)


def kernel(x_l0, x_l1, x_l2, x_l3, params, knn_idx):
    x0 = x_l0.reshape(_N, _C)
    x1 = x_l1.reshape(_N, _C)
    x2 = x_l2.reshape(_N, _C)
    x3 = x_l3.reshape(_N, _C)
    knn_flat = knn_idx.reshape(-1).astype(jnp.int32)

    qp0, qp1, qp2, qt0, qt1, qt2 = _tc_proj(x0, x1, x2, params['blocks'])
    g = _sc_gather(knn_flat, x3).reshape(_N, _K, _C)
    out = _tc_attn(g, (qt0, qt1, qt2), (qp0, qp1, qp2), x3, params)
    return out.reshape(1, _N, _C)


# speedup vs baseline: 5.7393x; 1.6274x over previous
import functools

import jax
import jax.numpy as jnp
from jax import lax
from jax.experimental import pallas as pl
from jax.experimental.pallas import tpu as pltpu
from jax.experimental.pallas import tpu_sc as plsc

_N = 10000
_C = 256
_K = 16
_T = 400
_NT = _N // _T
_CH = 200
_NCHUNK = (_N * _K) // _CH
_NW = 32
_ITERS = (_NCHUNK + _NW - 1) // _NW
_RSQRT_C = 1.0 / (_C ** 0.5)
_GATE_PAD = -1e30


def _ln(x, g, b, eps=1e-5):
    mu = jnp.mean(x, axis=-1, keepdims=True)
    var = jnp.mean((x - mu) ** 2, axis=-1, keepdims=True)
    return (x - mu) * lax.rsqrt(var + eps) * g + b


def _proj_body(x0, x1, x2,
               wq0, bq0, wk0, wq1, bq1, wk1, wq2, bq2, wk2,
               qp0, qp1, qp2, qt0, qt1, qt2):
    for x, wq, bq, wk, qp, qt in (
        (x0, wq0, bq0, wk0, qp0, qt0),
        (x1, wq1, bq1, wk1, qp1, qt1),
        (x2, wq2, bq2, wk2, qp2, qt2),
    ):
        q = jnp.dot(x[...], wq[...], preferred_element_type=jnp.float32) + bq[...]
        qp[...] = q
        qt[...] = lax.dot_general(
            q, wk[...], (((1,), (1,)), ((), ())),
            preferred_element_type=jnp.float32) * _RSQRT_C


def _tc_proj(x0, x1, x2, blocks):
    row = pl.BlockSpec((_T, _C), lambda i: (i, 0))
    wspec = pl.BlockSpec((_C, _C), lambda i: (0, 0))
    bspec = pl.BlockSpec((1, _C), lambda i: (0, 0))
    out = jax.ShapeDtypeStruct((_N, _C), jnp.float32)
    args = [x0, x1, x2]
    in_specs = [row, row, row]
    for blk in blocks:
        args += [blk['Wq'], blk['bq'].reshape(1, _C), blk['Wk']]
        in_specs += [wspec, bspec, wspec]
    return pl.pallas_call(
        _proj_body,
        grid=(_NT,),
        in_specs=in_specs,
        out_specs=[row] * 6,
        out_shape=[out] * 6,
    )(*args)


def _sc_gather_body(knn_hbm, x_hbm, g_hbm, idx_v, rows_v, sem):
    wid = lax.axis_index("s") * 2 + lax.axis_index("c")

    def step(t, carry):
        ch = wid + _NW * t

        @pl.when(ch < _NCHUNK)
        def _():
            pltpu.sync_copy(knn_hbm.at[pl.ds(ch * _CH, _CH)], idx_v)
            pltpu.async_copy(x_hbm.at[idx_v], rows_v, sem).wait()
            pltpu.sync_copy(rows_v, g_hbm.at[pl.ds(ch * _CH, _CH)])

        return carry

    lax.fori_loop(0, _ITERS, step, 0)


def _sc_gather(knn_flat, x):
    mesh = plsc.VectorSubcoreMesh(core_axis_name="c", subcore_axis_name="s")
    f = functools.partial(
        pl.kernel,
        out_type=jax.ShapeDtypeStruct((_N * _K, _C), jnp.float32),
        mesh=mesh,
        scratch_types=[
            pltpu.VMEM((_CH,), jnp.int32),
            pltpu.VMEM((_CH, _C), jnp.float32),
            pltpu.SemaphoreType.DMA,
        ],
    )(_sc_gather_body)
    return f(knn_flat, x)


def _attn_body(g, qt0, qt1, qt2, qp0, qp1, qp2, x3, bones,
               wv0, bv0, lg0, lb0, wv1, bv1, lg1, lb1, wv2, bv2, lg2, lb2,
               w1, b1, w2p, b2p, lgf, lbf, out):
    gk = [g[k] for k in range(_K)]
    x = x3[...]
    bo = bones[...]
    outs = []
    for qt, qp, wv, bv, lg, lb in (
        (qt0, qp0, wv0, bv0, lg0, lb0),
        (qt1, qp1, wv1, bv1, lg1, lb1),
        (qt2, qp2, wv2, bv2, lg2, lb2),
    ):
        q = qt[...]
        r = jnp.concatenate([gv * q for gv in gk], axis=-1)
        s = jnp.dot(r, bo, preferred_element_type=jnp.float32)
        m = jnp.max(s, axis=-1, keepdims=True)
        e = jnp.exp(s - m)
        inv = 1.0 / jnp.sum(e, axis=-1, keepdims=True)
        sv = e[:, 0:1] * gk[0]
        for k in range(1, _K):
            sv = sv + e[:, k:k + 1] * gk[k]
        sv = sv * inv
        o = jnp.dot(sv, wv[...], preferred_element_type=jnp.float32) + bv[...]
        outs.append(_ln(o + qp[...], lg[...], lb[...]))

    h = jnp.maximum(jnp.dot(x, w1[...], preferred_element_type=jnp.float32)
                    + b1[...], 0.0)
    logits = jnp.dot(h, w2p[...], preferred_element_type=jnp.float32) + b2p[...]
    lm = jnp.max(logits, axis=-1, keepdims=True)
    le = jnp.exp(logits - lm)
    linv = 1.0 / jnp.sum(le, axis=-1, keepdims=True)
    mix = (le[:, 0:1] * outs[0] + le[:, 1:2] * outs[1]
           + le[:, 2:3] * outs[2]) * linv
    out[...] = _ln(mix + x, lgf[...], lbf[...])


def _tc_attn(g, qts, qps, x3, params):
    row = pl.BlockSpec((_T, _C), lambda i: (i, 0))
    g3 = pl.BlockSpec((_K, _T, _C), lambda i: (0, i, 0))
    bones_spec = pl.BlockSpec((_K * _C, _K), lambda i: (0, 0))
    wspec = pl.BlockSpec((_C, _C), lambda i: (0, 0))
    bspec = pl.BlockSpec((1, _C), lambda i: (0, 0))
    hspec = pl.BlockSpec((_C, _C // 2), lambda i: (0, 0))
    h1spec = pl.BlockSpec((1, _C // 2), lambda i: (0, 0))
    gspec = pl.BlockSpec((_C // 2, _C // 2), lambda i: (0, 0))
    g1spec = pl.BlockSpec((1, _C // 2), lambda i: (0, 0))

    gate = params['gate']
    w2p = jnp.zeros((_C // 2, _C // 2), jnp.float32).at[:, :3].set(gate['W2'])
    b2p = jnp.full((_C // 2,), _GATE_PAD, jnp.float32).at[:3].set(gate['b2'])
    bones = jnp.kron(jnp.eye(_K, dtype=jnp.float32),
                     jnp.ones((_C, 1), jnp.float32))

    args = [g, qts[0], qts[1], qts[2], qps[0], qps[1], qps[2], x3, bones]
    in_specs = [g3, row, row, row, row, row, row, row, bones_spec]
    for blk in params['blocks']:
        args += [blk['Wv'], blk['bv'].reshape(1, _C),
                 blk['ln_g'].reshape(1, _C), blk['ln_b'].reshape(1, _C)]
        in_specs += [wspec, bspec, bspec, bspec]
    args += [gate['W1'], gate['b1'].reshape(1, _C // 2), w2p,
             b2p.reshape(1, _C // 2),
             params['ln_g'].reshape(1, _C), params['ln_b'].reshape(1, _C)]
    in_specs += [hspec, h1spec, gspec, g1spec, bspec, bspec]

    return pl.pallas_call(
        _attn_body,
        grid=(_NT,),
        in_specs=in_specs,
        out_specs=row,
        out_shape=jax.ShapeDtypeStruct((_N, _C), jnp.float32),
    )(*args)


def kernel(x_l0, x_l1, x_l2, x_l3, params, knn_idx):
    x0 = x_l0.reshape(_N, _C)
    x1 = x_l1.reshape(_N, _C)
    x2 = x_l2.reshape(_N, _C)
    x3 = x_l3.reshape(_N, _C)
    knn_flat = knn_idx.astype(jnp.int32).T.reshape(-1)

    qp0, qp1, qp2, qt0, qt1, qt2 = _tc_proj(x0, x1, x2, params['blocks'])
    g = _sc_gather(knn_flat, x3).reshape(_K, _N, _C)
    out = _tc_attn(g, (qt0, qt1, qt2), (qp0, qp1, qp2), x3, params)
    return out.reshape(1, _N, _C)
